# Initial kernel scaffold; baseline (speedup 1.0000x reference)
import jax
import jax.numpy as jnp
from jax.experimental import pallas as pl


def kernel(affinity_scores, num_detections, num_tracklets):
    raise NotImplementedError("write your pallas kernel here")


# speedup vs baseline: 1.1247x; 1.1247x over previous
import functools

import jax
import jax.numpy as jnp
from jax import lax
from jax.experimental import pallas as pl
from jax.experimental.pallas import tpu as pltpu
from jax.experimental.pallas import tpu_sc as plsc

LAMB = 10.0
N_ITERS = 20
EPS = 1e-12
B, T, D = 16, 512, 512
N = T * D


def _sinkhorn_body(t_ref, d_ref, aff_ref, p_ref, rm_ref):
    b = pl.program_id(0)
    t = t_ref[b]
    d = d_ref[b]
    tf = t.astype(jnp.float32)
    df = d.astype(jnp.float32)

    aff = aff_ref[0]
    row_i = lax.broadcasted_iota(jnp.int32, (T, D), 0)
    col_j = lax.broadcasted_iota(jnp.int32, (T, D), 1)
    real = (row_i < t) & (col_j < d)
    active = (row_i <= t) & (col_j <= d)
    K = jnp.where(active, jnp.where(real, jnp.exp(LAMB * aff), 1.0), 0.0)
    KT = K.T

    rows = lax.broadcasted_iota(jnp.int32, (T, 1), 0)
    rs = jnp.where(rows < t, 1.0, jnp.where(rows == t, tf, 0.0))
    cs = jnp.where(rows < d, 1.0, jnp.where(rows == d, df, 0.0))

    v0 = jnp.ones((T, 1), jnp.float32)
    u0 = jnp.zeros((T, 1), jnp.float32)

    def body(_, carry):
        u, v = carry
        Kv = lax.dot_general(K, v, (((1,), (0,)), ((), ())),
                             preferred_element_type=jnp.float32)
        u = rs / (Kv + EPS)
        KTu = lax.dot_general(KT, u, (((1,), (0,)), ((), ())),
                              preferred_element_type=jnp.float32)
        v = cs / (KTu + EPS)
        return u, v

    u, v = lax.fori_loop(0, N_ITERS, body, (u0, v0))

    vT = v.reshape(1, T)
    P = u * (K * vT)
    rm = jnp.max(P, axis=1, keepdims=True)
    p_ref[0] = jnp.where(real, P, 0.0)
    rm_ref[0] = rm.reshape(1, T)


def _sinkhorn_dense(aff, t_arr, d_arr):
    return pl.pallas_call(
        _sinkhorn_body,
        grid=(B,),
        in_specs=[
            pl.BlockSpec(memory_space=pltpu.SMEM),
            pl.BlockSpec(memory_space=pltpu.SMEM),
            pl.BlockSpec((1, T, D), lambda b: (b, 0, 0)),
        ],
        out_specs=[
            pl.BlockSpec((1, T, D), lambda b: (b, 0, 0)),
            pl.BlockSpec((1, 1, T), lambda b: (b, 0, 0)),
        ],
        out_shape=[
            jax.ShapeDtypeStruct((B, T, D), jnp.float32),
            jax.ShapeDtypeStruct((B, 1, T), jnp.float32),
        ],
    )(t_arr, d_arr, aff)


ZCHUNK = 4096
DUMP_OFF = 261632


def _sc_pack(p_hbm, rm_hbm, t_hbm, d_hbm, s_out, a_out,
             p_row, a_row, idx_buf, rm_row, zbuf, td_vec, sem, zsem):
    c = lax.axis_index("c")
    s = lax.axis_index("s")
    b = c * 8 + s // 2
    half = s % 2

    lane = lax.iota(jnp.int32, 16)

    zvec = jnp.zeros((16,), jnp.float32)

    def zb_body(i, _):
        zbuf[pl.ds(i * 16, 16)] = zvec
        return 0

    lax.fori_loop(0, ZCHUNK // 16, zb_body, 0)

    base = b * N + half * (N // 2)
    nchunks = (N // 2) // ZCHUNK
    for k in range(nchunks):
        pltpu.async_copy(zbuf, s_out.at[pl.ds(base + k * ZCHUNK, ZCHUNK)], zsem)
    for k in range(nchunks):
        pltpu.async_copy(zbuf, a_out.at[pl.ds(base + k * ZCHUNK, ZCHUNK)], zsem)
    for _ in range(2 * nchunks):
        pltpu.make_async_copy(zbuf, s_out.at[pl.ds(base, ZCHUNK)], zsem).wait()
    plsc.subcore_barrier()

    pltpu.sync_copy(t_hbm, td_vec.at[0, pl.ds(0, 16)])
    pltpu.sync_copy(d_hbm, td_vec.at[1, pl.ds(0, 16)])
    t = td_vec[0, pl.ds(b, 16)][0]
    d = td_vec[1, pl.ds(b, 16)][0]

    pltpu.sync_copy(rm_hbm.at[pl.ds(b * T, T)], rm_row.at[pl.ds(0, T)])

    h1 = (t + 1) // 2
    lo = jnp.where(half == 0, 0, h1)
    hi = jnp.where(half == 0, h1, t)

    dump = b * N + DUMP_OFF

    def row_body(r, _):
        pltpu.sync_copy(p_hbm.at[pl.ds((b * T + r) * D, D)], p_row)
        rm = jnp.full((16,), rm_row[pl.ds(r, 16)][0], jnp.float32)
        rd = b * N + r * d
        for j4 in range(4):
            for k in range(8):
                c0 = j4 * 128 + k * 16
                cvec = lane + c0
                p = p_row[pl.ds(c0, 16)]
                a_row[pl.ds(c0, 16)] = jnp.where(p >= rm, 1.0, 0.0)
                idx_buf[j4, pl.ds(k * 16, 16)] = jnp.where(
                    cvec < d, rd + cvec, dump + (cvec - j4 * 128))
        for j4 in range(4):
            @pl.when(j4 * 128 < d)
            def _():
                pltpu.async_copy(p_row.at[pl.ds(j4 * 128, 128)],
                                 s_out.at[idx_buf.at[j4]], sem)
                pltpu.async_copy(a_row.at[pl.ds(j4 * 128, 128)],
                                 a_out.at[idx_buf.at[j4]], sem)
        for j4 in range(4):
            @pl.when(j4 * 128 < d)
            def _():
                pltpu.make_async_copy(p_row.at[pl.ds(0, 128)],
                                      s_out.at[idx_buf.at[j4]], sem).wait()
                pltpu.make_async_copy(a_row.at[pl.ds(0, 128)],
                                      a_out.at[idx_buf.at[j4]], sem).wait()
        return 0

    lax.fori_loop(lo, hi, row_body, 0)


def _ragged_pack(p_flat, rm_flat, t_arr, d_arr):
    mesh = plsc.VectorSubcoreMesh(core_axis_name="c", subcore_axis_name="s")
    fn = functools.partial(
        pl.kernel,
        mesh=mesh,
        out_type=[
            jax.ShapeDtypeStruct((B * N,), jnp.float32),
            jax.ShapeDtypeStruct((B * N,), jnp.float32),
        ],
        scratch_types=[
            pltpu.VMEM((D,), jnp.float32),
            pltpu.VMEM((D,), jnp.float32),
            pltpu.VMEM((4, 128), jnp.int32),
            pltpu.VMEM((T + 16,), jnp.float32),
            pltpu.VMEM((ZCHUNK,), jnp.float32),
            pltpu.VMEM((2, 32), jnp.int32),
            pltpu.SemaphoreType.DMA,
            pltpu.SemaphoreType.DMA,
        ],
    )(_sc_pack)
    return fn(p_flat, rm_flat, t_arr, d_arr)


def kernel(affinity_scores, num_detections, num_tracklets):
    p, rm = _sinkhorn_dense(affinity_scores, num_tracklets, num_detections)
    s_flat, a_flat = _ragged_pack(
        p.reshape(B * T * D), rm.reshape(B * T), num_tracklets, num_detections)
    sinkhorn_dense = s_flat.reshape(B, N)
    assignment_dense = a_flat.reshape(B, N).astype(jnp.bool_)
    return sinkhorn_dense, assignment_dense


# speedup vs baseline: 3.0622x; 2.7227x over previous
import functools

import jax
import jax.numpy as jnp
from jax import lax
from jax.experimental import pallas as pl
from jax.experimental.pallas import tpu as pltpu
from jax.experimental.pallas import tpu_sc as plsc

LAMB = 10.0
N_ITERS = 20
EPS = 1e-12
B, T, D = 16, 512, 512
N = T * D


def _sinkhorn_body(t_ref, d_ref, aff_ref, p_ref, rm_ref):
    b = pl.program_id(0)
    t = t_ref[b]
    d = d_ref[b]
    tf = t.astype(jnp.float32)
    df = d.astype(jnp.float32)

    aff = aff_ref[0]
    row_i = lax.broadcasted_iota(jnp.int32, (T, D), 0)
    col_j = lax.broadcasted_iota(jnp.int32, (T, D), 1)
    real = (row_i < t) & (col_j < d)
    active = (row_i <= t) & (col_j <= d)
    K = jnp.where(active, jnp.where(real, jnp.exp(LAMB * aff), 1.0), 0.0)
    KT = K.T

    rows = lax.broadcasted_iota(jnp.int32, (T, 1), 0)
    rs = jnp.where(rows < t, 1.0, jnp.where(rows == t, tf, 0.0))
    cs = jnp.where(rows < d, 1.0, jnp.where(rows == d, df, 0.0))

    v0 = jnp.ones((T, 1), jnp.float32)
    u0 = jnp.zeros((T, 1), jnp.float32)

    def body(_, carry):
        u, v = carry
        Kv = lax.dot_general(K, v, (((1,), (0,)), ((), ())),
                             preferred_element_type=jnp.float32)
        u = rs / (Kv + EPS)
        KTu = lax.dot_general(KT, u, (((1,), (0,)), ((), ())),
                              preferred_element_type=jnp.float32)
        v = cs / (KTu + EPS)
        return u, v

    u, v = lax.fori_loop(0, N_ITERS, body, (u0, v0))

    vT = v.reshape(1, T)
    P = u * (K * vT)
    rm = jnp.max(P, axis=1, keepdims=True)
    p_ref[0] = jnp.where(real, P, 0.0)
    rm_ref[0] = rm.reshape(1, T)


def _sinkhorn_dense(aff, t_arr, d_arr):
    return pl.pallas_call(
        _sinkhorn_body,
        grid=(B,),
        in_specs=[
            pl.BlockSpec(memory_space=pltpu.SMEM),
            pl.BlockSpec(memory_space=pltpu.SMEM),
            pl.BlockSpec((1, T, D), lambda b: (b, 0, 0)),
        ],
        out_specs=[
            pl.BlockSpec((1, T, D), lambda b: (b, 0, 0)),
            pl.BlockSpec((1, 1, T), lambda b: (b, 0, 0)),
        ],
        out_shape=[
            jax.ShapeDtypeStruct((B, T, D), jnp.float32),
            jax.ShapeDtypeStruct((B, 1, T), jnp.float32),
        ],
    )(t_arr, d_arr, aff)


TILE = 4096
NT_TILES = N // TILE
UNITS = B * NT_TILES
NW = 32
UPW = UNITS // NW


def _sc_pack(p_hbm, rm_hbm, t_hbm, d_hbm, rec_hbm, s_out, a_out,
             idx_p, idx_r, p_tile, rm_tile, td_vec, rec_vec, psem, rsem):
    c_ax = lax.axis_index("c")
    s_ax = lax.axis_index("s")
    w = c_ax * 16 + s_ax
    lane = lax.iota(jnp.int32, 16)

    pltpu.sync_copy(t_hbm, td_vec.at[0, pl.ds(0, 16)])
    pltpu.sync_copy(d_hbm, td_vec.at[1, pl.ds(0, 16)])
    pltpu.sync_copy(rec_hbm, rec_vec.at[pl.ds(0, 16)])

    def unit_body(uu, _):
        unit = uu * NW + w
        b = unit // NT_TILES
        tile = unit - b * NT_TILES
        k0 = tile * TILE
        t = td_vec[0, pl.ds(b, 16)][0]
        d = td_vec[1, pl.ds(b, 16)][0]
        bP = b * N
        bR = b * T
        rec = rec_vec[pl.ds(b, 16)][0]

        def row_body(row, _):
            kbase = k0 + row * 128
            for kk in range(8):
                kc = kbase + kk * 16 + lane
                r0 = (kc.astype(jnp.float32) * rec).astype(jnp.int32)
                r0 = r0 - jnp.where(r0 * d > kc, 1, 0)
                r0 = r0 + jnp.where((r0 + 1) * d <= kc, 1, 0)
                r = jnp.minimum(r0, t)
                cc = jnp.where(r < t, kc - r * d, 0)
                idx_p[row, pl.ds(kk * 16, 16)] = bP + r * D + cc
                idx_r[row, pl.ds(kk * 16, 16)] = bR + r
            return 0

        lax.fori_loop(0, 32, row_body, 0)

        def fire_body(row, _):
            pltpu.async_copy(p_hbm.at[idx_p.at[row]], p_tile.at[row], psem)
            pltpu.async_copy(rm_hbm.at[idx_r.at[row]], rm_tile.at[row], rsem)
            return 0

        lax.fori_loop(0, 32, fire_body, 0)

        def drain_body(row, _):
            pltpu.make_async_copy(p_hbm.at[idx_p.at[row]],
                                  p_tile.at[row], psem).wait()
            pltpu.make_async_copy(rm_hbm.at[idx_r.at[row]],
                                  rm_tile.at[row], rsem).wait()
            return 0

        lax.fori_loop(0, 32, drain_body, 0)

        def a_body(row, _):
            for kk in range(8):
                p = p_tile[row, pl.ds(kk * 16, 16)]
                rm = rm_tile[row, pl.ds(kk * 16, 16)]
                rm_tile[row, pl.ds(kk * 16, 16)] = jnp.where(p >= rm, 1.0, 0.0)
            return 0

        lax.fori_loop(0, 32, a_body, 0)
        pltpu.sync_copy(p_tile, s_out.at[unit])
        pltpu.sync_copy(rm_tile, a_out.at[unit])
        return 0

    lax.fori_loop(0, UPW, unit_body, 0)


def _ragged_pack(p_flat, rm_flat, t_arr, d_arr):
    mesh = plsc.VectorSubcoreMesh(core_axis_name="c", subcore_axis_name="s")
    fn = functools.partial(
        pl.kernel,
        mesh=mesh,
        out_type=[
            jax.ShapeDtypeStruct((UNITS, 32, 128), jnp.float32),
            jax.ShapeDtypeStruct((UNITS, 32, 128), jnp.float32),
        ],
        scratch_types=[
            pltpu.VMEM((32, 128), jnp.int32),
            pltpu.VMEM((32, 128), jnp.int32),
            pltpu.VMEM((32, 128), jnp.float32),
            pltpu.VMEM((32, 128), jnp.float32),
            pltpu.VMEM((2, 32), jnp.int32),
            pltpu.VMEM((32,), jnp.float32),
            pltpu.SemaphoreType.DMA,
            pltpu.SemaphoreType.DMA,
        ],
    )(_sc_pack)
    rec_arr = 1.0 / d_arr.astype(jnp.float32)
    return fn(p_flat, rm_flat, t_arr, d_arr, rec_arr)


def kernel(affinity_scores, num_detections, num_tracklets):
    p, rm = _sinkhorn_dense(affinity_scores, num_tracklets, num_detections)
    s_tiles, a_tiles = _ragged_pack(
        p.reshape(B * T * D), rm.reshape(B * T), num_tracklets, num_detections)
    sinkhorn_dense = s_tiles.reshape(B, N)
    assignment_dense = a_tiles.reshape(B, N).astype(jnp.bool_)
    return sinkhorn_dense, assignment_dense


# speedup vs baseline: 13.7982x; 4.5059x over previous
import functools

import jax
import jax.numpy as jnp
from jax import lax
from jax.experimental import pallas as pl
from jax.experimental.pallas import tpu as pltpu
from jax.experimental.pallas import tpu_sc as plsc

LAMB = 10.0
N_ITERS = 20
EPS = 1e-12
B, T, D = 16, 512, 512
N = T * D


def _sinkhorn_body(t_ref, d_ref, aff_ref, p_ref, rm_ref):
    b = pl.program_id(0)
    t = t_ref[b]
    d = d_ref[b]
    tf = t.astype(jnp.float32)
    df = d.astype(jnp.float32)

    aff = aff_ref[0]
    row_i = lax.broadcasted_iota(jnp.int32, (T, D), 0)
    col_j = lax.broadcasted_iota(jnp.int32, (T, D), 1)
    real = (row_i < t) & (col_j < d)
    active = (row_i <= t) & (col_j <= d)
    K = jnp.where(active, jnp.where(real, jnp.exp(LAMB * aff), 1.0), 0.0)
    KT = K.T

    rows = lax.broadcasted_iota(jnp.int32, (T, 1), 0)
    rs = jnp.where(rows < t, 1.0, jnp.where(rows == t, tf, 0.0))
    cs = jnp.where(rows < d, 1.0, jnp.where(rows == d, df, 0.0))

    v0 = jnp.ones((T, 1), jnp.float32)
    u0 = jnp.zeros((T, 1), jnp.float32)

    def body(_, carry):
        u, v = carry
        Kv = lax.dot_general(K, v, (((1,), (0,)), ((), ())),
                             preferred_element_type=jnp.float32)
        u = rs / (Kv + EPS)
        KTu = lax.dot_general(KT, u, (((1,), (0,)), ((), ())),
                              preferred_element_type=jnp.float32)
        v = cs / (KTu + EPS)
        return u, v

    u, v = lax.fori_loop(0, N_ITERS, body, (u0, v0))

    vT = v.reshape(1, T)
    P = u * (K * vT)
    rm = jnp.max(P, axis=1, keepdims=True)
    p_ref[0] = jnp.where(real, P, 0.0)
    rm_ref[0] = rm.reshape(1, T)


def _sinkhorn_dense(aff, t_arr, d_arr):
    return pl.pallas_call(
        _sinkhorn_body,
        grid=(B,),
        in_specs=[
            pl.BlockSpec(memory_space=pltpu.SMEM),
            pl.BlockSpec(memory_space=pltpu.SMEM),
            pl.BlockSpec((1, T, D), lambda b: (b, 0, 0)),
        ],
        out_specs=[
            pl.BlockSpec((1, T, D), lambda b: (b, 0, 0)),
            pl.BlockSpec((1, 1, T), lambda b: (b, 0, 0)),
        ],
        out_shape=[
            jax.ShapeDtypeStruct((B, T, D), jnp.float32),
            jax.ShapeDtypeStruct((B, 1, T), jnp.float32),
        ],
    )(t_arr, d_arr, aff)


TILE = 4096
NT_TILES = N // TILE
UNITS = B * NT_TILES
NW = 32
UPW = UNITS // NW


def _sc_pack(p_hbm, rm_hbm, t_hbm, d_hbm, rec_hbm, s_out, a_out,
             idx_p, idx_r, p_tile, rm_tile, zbuf, td_vec, rec_vec,
             psem, rsem):
    c_ax = lax.axis_index("c")
    s_ax = lax.axis_index("s")
    w = c_ax * 16 + s_ax
    lane = lax.iota(jnp.int32, 16)

    pltpu.sync_copy(t_hbm, td_vec.at[0, pl.ds(0, 16)])
    pltpu.sync_copy(d_hbm, td_vec.at[1, pl.ds(0, 16)])
    pltpu.sync_copy(rec_hbm, rec_vec.at[pl.ds(0, 16)])

    zvec = jnp.zeros((16,), jnp.float32)

    def zb_body(row, _):
        for kk in range(8):
            zbuf[row, pl.ds(kk * 16, 16)] = zvec
        return 0

    lax.fori_loop(0, 32, zb_body, 0)

    def unit_body(uu, _):
        unit = uu * NW + w
        b = unit // NT_TILES
        tile = unit - b * NT_TILES
        k0 = tile * TILE
        t = td_vec[0, pl.ds(b, 16)][0]
        d = td_vec[1, pl.ds(b, 16)][0]
        bP = b * N
        bR = b * T
        rec = rec_vec[pl.ds(b, 16)][0]
        nvalid = jnp.clip(t * d - k0, 0, TILE)
        n128 = (nvalid + 127) >> 7

        @pl.when(n128 == 0)
        def _():
            pltpu.sync_copy(zbuf, s_out.at[unit])
            pltpu.sync_copy(zbuf, a_out.at[unit])

        @pl.when(n128 > 0)
        def _():
            def row_body(row, _):
                kbase = k0 + row * 128
                for kk in range(8):
                    kc = kbase + kk * 16 + lane
                    r0 = (kc.astype(jnp.float32) * rec).astype(jnp.int32)
                    r0 = r0 - jnp.where(r0 * d > kc, 1, 0)
                    r0 = r0 + jnp.where((r0 + 1) * d <= kc, 1, 0)
                    r = jnp.minimum(r0, t)
                    cc = jnp.where(r < t, kc - r * d, 0)
                    idx_p[row, pl.ds(kk * 16, 16)] = bP + r * D + cc
                    idx_r[row, pl.ds(kk * 16, 16)] = bR + r
                return 0

            lax.fori_loop(0, n128, row_body, 0)

            def ztail_body(row, _):
                for kk in range(8):
                    p_tile[row, pl.ds(kk * 16, 16)] = zvec
                    rm_tile[row, pl.ds(kk * 16, 16)] = zvec
                return 0

            lax.fori_loop(n128, 32, ztail_body, 0)

            def fire_body(row, _):
                pltpu.async_copy(p_hbm.at[idx_p.at[row]], p_tile.at[row], psem)
                pltpu.async_copy(rm_hbm.at[idx_r.at[row]], rm_tile.at[row],
                                 rsem)
                return 0

            lax.fori_loop(0, n128, fire_body, 0)

            def drain_body(row, _):
                pltpu.make_async_copy(p_hbm.at[idx_p.at[row]],
                                      p_tile.at[row], psem).wait()
                pltpu.make_async_copy(rm_hbm.at[idx_r.at[row]],
                                      rm_tile.at[row], rsem).wait()
                return 0

            lax.fori_loop(0, n128, drain_body, 0)

            def a_body(row, _):
                for kk in range(8):
                    p = p_tile[row, pl.ds(kk * 16, 16)]
                    rm = rm_tile[row, pl.ds(kk * 16, 16)]
                    rm_tile[row, pl.ds(kk * 16, 16)] = jnp.where(
                        p >= rm, 1.0, 0.0)
                return 0

            lax.fori_loop(0, n128, a_body, 0)
            pltpu.sync_copy(p_tile, s_out.at[unit])
            pltpu.sync_copy(rm_tile, a_out.at[unit])
        return 0

    lax.fori_loop(0, UPW, unit_body, 0)


def _ragged_pack(p_flat, rm_flat, t_arr, d_arr):
    mesh = plsc.VectorSubcoreMesh(core_axis_name="c", subcore_axis_name="s")
    fn = functools.partial(
        pl.kernel,
        mesh=mesh,
        out_type=[
            jax.ShapeDtypeStruct((UNITS, 32, 128), jnp.float32),
            jax.ShapeDtypeStruct((UNITS, 32, 128), jnp.float32),
        ],
        scratch_types=[
            pltpu.VMEM((32, 128), jnp.int32),
            pltpu.VMEM((32, 128), jnp.int32),
            pltpu.VMEM((32, 128), jnp.float32),
            pltpu.VMEM((32, 128), jnp.float32),
            pltpu.VMEM((32, 128), jnp.float32),
            pltpu.VMEM((2, 32), jnp.int32),
            pltpu.VMEM((32,), jnp.float32),
            pltpu.SemaphoreType.DMA,
            pltpu.SemaphoreType.DMA,
        ],
    )(_sc_pack)
    rec_arr = 1.0 / d_arr.astype(jnp.float32)
    return fn(p_flat, rm_flat, t_arr, d_arr, rec_arr)


def kernel(affinity_scores, num_detections, num_tracklets):
    p, rm = _sinkhorn_dense(affinity_scores, num_tracklets, num_detections)
    s_tiles, a_tiles = _ragged_pack(
        p.reshape(B * T * D), rm.reshape(B * T), num_tracklets, num_detections)
    sinkhorn_dense = s_tiles.reshape(B, N)
    assignment_dense = a_tiles.reshape(B, N).astype(jnp.bool_)
    return sinkhorn_dense, assignment_dense
